# Initial kernel scaffold; baseline (speedup 1.0000x reference)
import jax
import jax.numpy as jnp
from jax.experimental import pallas as pl


def kernel(x, edge_index, W1, b1, W2, b2):
    raise NotImplementedError("write your pallas kernel here")


# speedup vs baseline: 7.8844x; 7.8844x over previous
import functools

import jax
import jax.numpy as jnp
from jax import lax
from jax.experimental import pallas as pl
from jax.experimental.pallas import tpu as pltpu
from jax.experimental.pallas import tpu_sc as plsc

K_PROP = 10
ALPHA = 0.1
NS = 16
NC = 2
CH = 128
KB = 3
RB = 128
BN = 2000


def _mlp_body(x_ref, w1_ref, b1_ref, w2_ref, b2_ref, o_ref):
    h1 = jnp.dot(x_ref[...], w1_ref[...], preferred_element_type=jnp.float32)
    h1 = jnp.maximum(h1 + b1_ref[...], 0.0)
    o_ref[...] = (
        jnp.dot(h1, w2_ref[...], preferred_element_type=jnp.float32)
        + b2_ref[...]
    )


def _prep_body(h_ref, d_ref, u0_ref, c32_ref, g_ref, sq_ref):
    deg = d_ref[:, 0:1] + 1.0
    dinv = lax.rsqrt(deg)
    h = h_ref[...]
    u0_ref[...] = dinv * h
    c32_ref[...] = jnp.broadcast_to((1.0 - ALPHA) * dinv * dinv, h.shape)
    g_ref[...] = (ALPHA * dinv) * h
    sq_ref[...] = jnp.broadcast_to(jnp.sqrt(deg), d_ref.shape)


def _final_body(u_ref, sq_ref, o_ref):
    z = u_ref[...] * sq_ref[:, 0:1]
    m = jnp.max(z, axis=1, keepdims=True)
    zs = z - m
    o_ref[...] = zs - jnp.log(jnp.sum(jnp.exp(zs), axis=1, keepdims=True))


def _compute_offsets(colv, offv, base, half, trash):
    for b in range(KB):
        for k in range(CH // 16):
            sl = pl.ds(16 * k, 16)
            cv = colv[0, b, sl]
            off = cv - base
            ok = (off >= 0) & (off < half)
            offv[0, b, sl] = jnp.where(ok, off, trash)


def _make_deg_kernel(n, half, aggr, gpt):
    trash = half
    hpt = (-(-half // NS) + 7) // 8 * 8
    nch = -(-hpt // RB)
    mesh = plsc.VectorSubcoreMesh(core_axis_name="c", subcore_axis_name="s")

    @functools.partial(
        pl.kernel,
        out_type=jax.ShapeDtypeStruct((n, 16), jnp.float32),
        mesh=mesh,
        compiler_params=pltpu.CompilerParams(use_tc_tiling_on_sc=False),
        scratch_types=[
            pltpu.VMEM_SHARED((aggr, 16), jnp.float32),
            pltpu.VMEM((RB, 16), jnp.float32),
            pltpu.VMEM((1, KB, CH), jnp.int32),
            pltpu.VMEM((1, KB, CH), jnp.int32),
            pltpu.VMEM((RB, 16), jnp.float32),
        ],
    )
    def deg_kernel(col_hbm, deg_hbm, agg, fill, colv, offv, stage):
        c = lax.axis_index("c")
        s = lax.axis_index("s")
        base = c * half

        def fill_rows(val):
            def frow(r, _):
                fill[r, pl.ds(0, 16)] = jnp.full((16,), val, jnp.float32)
                return 0
            lax.fori_loop(0, RB, frow, 0)

        fill_rows(0.0)

        def zchunk(j, _):
            lo = jnp.minimum(s * hpt + j * RB, aggr - RB)
            pltpu.sync_copy(fill, agg.at[pl.ds(lo, RB)])
            return 0
        lax.fori_loop(0, nch + 1, zchunk, 0)
        fill_rows(1.0)
        plsc.subcore_barrier()

        def group(g, _):
            pltpu.sync_copy(col_hbm.at[pl.ds(s * gpt + g, 1)], colv)
            _compute_offsets(colv, offv, base, half, trash)
            for b in range(KB):
                pltpu.sync_copy(fill, agg.at[offv.at[0, b]], add=True)
            return 0
        lax.fori_loop(0, gpt, group, 0)
        plsc.subcore_barrier()

        def out_chunk(j, _):
            r = jnp.minimum(s * hpt + j * RB, half - RB)
            pltpu.sync_copy(agg.at[pl.ds(r, RB)], stage)
            pltpu.sync_copy(stage, deg_hbm.at[pl.ds(base + r, RB)])
            return 0
        lax.fori_loop(0, nch, out_chunk, 0)

    return deg_kernel


def _make_prop_kernel(n, half, aggr, gpt):
    trash = half
    hpt = (-(-half // NS) + 7) // 8 * 8
    nch = -(-hpt // RB)
    n_pairs = gpt // 2 - 1
    mesh = plsc.VectorSubcoreMesh(core_axis_name="c", subcore_axis_name="s")

    @functools.partial(
        pl.kernel,
        out_type=jax.ShapeDtypeStruct((n, 32), jnp.float32),
        mesh=mesh,
        compiler_params=pltpu.CompilerParams(use_tc_tiling_on_sc=False),
        scratch_types=[
            pltpu.VMEM_SHARED((aggr, 32), jnp.float32),
            pltpu.VMEM((1, KB, CH), jnp.int32),
            pltpu.VMEM((1, KB, CH), jnp.int32),
            pltpu.VMEM((1, KB, CH), jnp.int32),
            pltpu.VMEM((1, KB, CH), jnp.int32),
            pltpu.VMEM((1, KB, CH), jnp.int32),
            pltpu.VMEM((1, KB, CH), jnp.int32),
            pltpu.VMEM((KB, CH, 32), jnp.float32),
            pltpu.VMEM((KB, CH, 32), jnp.float32),
            pltpu.SemaphoreType.DMA,
            pltpu.SemaphoreType.DMA,
        ],
    )
    def prop_kernel(u_hbm, row_hbm, col_hbm, c32_hbm, g_hbm, out_hbm,
                    agg, colv0, colv1, offv0, offv1, rowv0, rowv1,
                    rows0, rows1, sem0, sem1):
        zbuf = rows0.at[0]
        bagg, bu = rows0.at[1], rows0.at[2]
        bc, bg = rows1.at[0], rows1.at[1]
        c = lax.axis_index("c")
        s = lax.axis_index("s")
        base = c * half

        def zrow(r, _):
            zbuf[r, pl.ds(0, 16)] = jnp.zeros((16,), jnp.float32)
            zbuf[r, pl.ds(16, 16)] = jnp.zeros((16,), jnp.float32)
            return 0
        lax.fori_loop(0, RB, zrow, 0)
        zc = zbuf

        def zchunk(j, _):
            lo = jnp.minimum(s * hpt + j * RB, aggr - RB)
            pltpu.sync_copy(zc, agg.at[pl.ds(lo, RB)])
            return 0
        lax.fori_loop(0, nch + 1, zchunk, 0)
        plsc.subcore_barrier()

        def fire(g, colv, offv, rowv, rows, sem):
            gg = s * gpt + g
            pltpu.sync_copy(col_hbm.at[pl.ds(gg, 1)], colv)
            pltpu.sync_copy(row_hbm.at[pl.ds(gg, 1)], rowv)
            _compute_offsets(colv, offv, base, half, trash)
            for b in range(KB):
                pltpu.async_copy(u_hbm.at[rowv.at[0, b]], rows.at[b], sem)

        def drain(offv, rowv, rows, sem):
            for b in range(KB):
                pltpu.make_async_copy(
                    u_hbm.at[rowv.at[0, b]], rows.at[b], sem).wait()
            for b in range(KB):
                pltpu.sync_copy(rows.at[b], agg.at[offv.at[0, b]], add=True)

        fire(0, colv0, offv0, rowv0, rows0, sem0)

        def pair(j, _):
            fire(2 * j + 1, colv1, offv1, rowv1, rows1, sem1)
            drain(offv0, rowv0, rows0, sem0)
            fire(2 * j + 2, colv0, offv0, rowv0, rows0, sem0)
            drain(offv1, rowv1, rows1, sem1)
            return 0
        lax.fori_loop(0, n_pairs, pair, 0)
        fire(gpt - 1, colv1, offv1, rowv1, rows1, sem1)
        drain(offv0, rowv0, rows0, sem0)
        drain(offv1, rowv1, rows1, sem1)
        plsc.subcore_barrier()

        def bchunk(j, _):
            r = jnp.minimum(s * hpt + j * RB, half - RB)
            lo = base + r
            pltpu.sync_copy(agg.at[pl.ds(r, RB)], bagg)
            pltpu.sync_copy(u_hbm.at[pl.ds(lo, RB)], bu)
            pltpu.sync_copy(c32_hbm.at[pl.ds(lo, RB)], bc)
            pltpu.sync_copy(g_hbm.at[pl.ds(lo, RB)], bg)

            def rrow(rr, _):
                for k in range(2):
                    sl = pl.ds(16 * k, 16)
                    bagg[rr, sl] = bc[rr, sl] * (bagg[rr, sl] + bu[rr, sl]) \
                        + bg[rr, sl]
                return 0
            lax.fori_loop(0, RB, rrow, 0)
            pltpu.sync_copy(bagg, out_hbm.at[pl.ds(lo, RB)])
            return 0
        lax.fori_loop(0, nch, bchunk, 0)

    return prop_kernel


def kernel(x, edge_index, W1, b1, W2, b2):
    n, ic = x.shape
    hc = W1.shape[1]
    oc = W2.shape[1]
    e = edge_index.shape[1]

    half = n // 2
    aggr = half + NS
    slab = KB * CH
    group_edges = NS * slab * 2
    ep = ((e + group_edges - 1) // group_edges) * group_edges
    nbs = ep // slab
    gpt = nbs // NS

    row = edge_index[0].astype(jnp.int32)
    col = edge_index[1].astype(jnp.int32)
    pad = ep - e
    row3 = jnp.concatenate(
        [row, jnp.zeros((pad,), jnp.int32)]).reshape(nbs, KB, CH)
    col3 = jnp.concatenate(
        [col, jnp.full((pad,), n, jnp.int32)]).reshape(nbs, KB, CH)

    grid = (n // BN,)
    h = pl.pallas_call(
        _mlp_body,
        grid=grid,
        in_specs=[
            pl.BlockSpec((BN, ic), lambda i: (i, 0)),
            pl.BlockSpec((ic, hc), lambda i: (0, 0)),
            pl.BlockSpec((1, hc), lambda i: (0, 0)),
            pl.BlockSpec((hc, oc), lambda i: (0, 0)),
            pl.BlockSpec((1, oc), lambda i: (0, 0)),
        ],
        out_specs=pl.BlockSpec((BN, oc), lambda i: (i, 0)),
        out_shape=jax.ShapeDtypeStruct((n, oc), jnp.float32),
    )(x, W1, b1.reshape(1, hc), W2, b2.reshape(1, oc))

    deg16 = _make_deg_kernel(n, half, aggr, gpt)(col3)

    u, c32, g, sq = pl.pallas_call(
        _prep_body,
        grid=grid,
        in_specs=[
            pl.BlockSpec((BN, oc), lambda i: (i, 0)),
            pl.BlockSpec((BN, 16), lambda i: (i, 0)),
        ],
        out_specs=[
            pl.BlockSpec((BN, oc), lambda i: (i, 0)),
            pl.BlockSpec((BN, oc), lambda i: (i, 0)),
            pl.BlockSpec((BN, oc), lambda i: (i, 0)),
            pl.BlockSpec((BN, 16), lambda i: (i, 0)),
        ],
        out_shape=[
            jax.ShapeDtypeStruct((n, oc), jnp.float32),
            jax.ShapeDtypeStruct((n, oc), jnp.float32),
            jax.ShapeDtypeStruct((n, oc), jnp.float32),
            jax.ShapeDtypeStruct((n, 16), jnp.float32),
        ],
    )(h, deg16)

    prop = _make_prop_kernel(n, half, aggr, gpt)
    for _ in range(K_PROP):
        u = prop(u, row3, col3, c32, g)

    out = pl.pallas_call(
        _final_body,
        grid=grid,
        in_specs=[
            pl.BlockSpec((BN, oc), lambda i: (i, 0)),
            pl.BlockSpec((BN, 16), lambda i: (i, 0)),
        ],
        out_specs=pl.BlockSpec((BN, oc), lambda i: (i, 0)),
        out_shape=jax.ShapeDtypeStruct((n, oc), jnp.float32),
    )(u, sq)
    return out
